# baseline (speedup 1.0000x reference)
import jax
import jax.numpy as jnp
from jax.experimental import pallas as pl

_B_BLOCK = 128


def _bcast_kernel(row_ref, o_ref):
    o_ref[...] = jnp.broadcast_to(row_ref[...], o_ref.shape)


def kernel(x, pos_embed_weight):
    B = x.shape[0]
    L, C = pos_embed_weight.shape
    row = pos_embed_weight.T.reshape(1, C * L)
    flat = pl.pallas_call(
        _bcast_kernel,
        grid=(B // _B_BLOCK,),
        in_specs=[pl.BlockSpec((1, C * L), lambda i: (0, 0))],
        out_specs=pl.BlockSpec((_B_BLOCK, C * L), lambda i: (i, 0)),
        out_shape=jax.ShapeDtypeStruct((B, C * L), jnp.float32),
    )(row)
    return flat.reshape(B, C, L)


# speedup vs baseline: 1.0061x; 1.0061x over previous
import jax
import jax.numpy as jnp
from jax.experimental import pallas as pl
from jax.experimental.pallas import tpu as pltpu

_R = 512
_NSEM = 8


def _bcast_kernel(row_ref, o_ref, buf_ref, sems):
    buf_ref[...] = jnp.broadcast_to(row_ref[...], buf_ref.shape)
    nchunk = o_ref.shape[0] // _R
    for q in range(nchunk):
        pltpu.make_async_copy(
            buf_ref, o_ref.at[pl.ds(q * _R, _R)], sems.at[q % _NSEM]
        ).start()
    for q in range(nchunk):
        pltpu.make_async_copy(
            buf_ref, o_ref.at[pl.ds(q * _R, _R)], sems.at[q % _NSEM]
        ).wait()


def kernel(x, pos_embed_weight):
    B = x.shape[0]
    L, C = pos_embed_weight.shape
    row = pos_embed_weight.T.reshape(1, C * L)
    flat = pl.pallas_call(
        _bcast_kernel,
        in_specs=[pl.BlockSpec(memory_space=pltpu.MemorySpace.VMEM)],
        out_specs=pl.BlockSpec(memory_space=pl.ANY),
        out_shape=jax.ShapeDtypeStruct((B, C * L), jnp.float32),
        scratch_shapes=[
            pltpu.VMEM((_R, C * L), jnp.float32),
            pltpu.SemaphoreType.DMA((_NSEM,)),
        ],
    )(row)
    return flat.reshape(B, C, L)


# speedup vs baseline: 1.0726x; 1.0662x over previous
import jax
import jax.numpy as jnp
from jax.experimental import pallas as pl
from jax.experimental.pallas import tpu as pltpu

_B_BLOCK = 128


def _bcast_kernel(w_ref, o_ref, wt_ref):
    @pl.when(pl.program_id(0) == 0)
    def _():
        wt_ref[...] = w_ref[...].T

    o_ref[...] = jnp.broadcast_to(wt_ref[...][None, :, :], o_ref.shape)


def kernel(x, pos_embed_weight):
    B = x.shape[0]
    L, C = pos_embed_weight.shape
    return pl.pallas_call(
        _bcast_kernel,
        grid=(B // _B_BLOCK,),
        in_specs=[pl.BlockSpec((L, C), lambda i: (0, 0))],
        out_specs=pl.BlockSpec((_B_BLOCK, C, L), lambda i: (i, 0, 0)),
        out_shape=jax.ShapeDtypeStruct((B, C, L), jnp.float32),
        scratch_shapes=[pltpu.VMEM((C, L), jnp.float32)],
    )(pos_embed_weight)


# speedup vs baseline: 9.1796x; 8.5582x over previous
import jax
import jax.numpy as jnp
from jax.experimental import pallas as pl

_B_BLOCK = 8192


def _bcast_kernel(w_ref, o_ref):
    l = pl.program_id(0)
    row = w_ref[pl.ds(l, 1), :]
    o_ref[...] = jnp.broadcast_to(row[:, None, :], o_ref.shape)


def kernel(x, pos_embed_weight):
    B = x.shape[0]
    L, C = pos_embed_weight.shape
    lbc = pl.pallas_call(
        _bcast_kernel,
        grid=(L, B // _B_BLOCK),
        in_specs=[pl.BlockSpec((L, C), lambda l, i: (0, 0))],
        out_specs=pl.BlockSpec((1, _B_BLOCK, C), lambda l, i: (l, i, 0)),
        out_shape=jax.ShapeDtypeStruct((L, B, C), jnp.float32),
    )(pos_embed_weight)
    return jnp.transpose(lbc, (1, 2, 0))
